# baseline (speedup 1.0000x reference)
import jax
import jax.numpy as jnp
from jax import lax
from jax.experimental import pallas as pl
from jax.experimental.pallas import tpu as pltpu
from jax.experimental.pallas import tpu_sc as plsc

_B, _S, _D, _T = 4, 4096, 2048, 64
_N = _T + _S
_SD, _TD, _ND = _S * _D, _T * _D, _N * _D
_NC, _NS = 2, 16
_W = _NC * _NS
_P = _W // _B
_C = 16384
_NT = _SD // _P // _C
_NBUF = 4
_NOUTER = _NT // _NBUF


def _al(x):
    return pl.multiple_of(x, _C)


def _sc_body(text, mask, labels, task, oe, om, ol,
             buf, tbuf, mbuf, lbuf, pbuf_m, pbuf_l,
             isem, osem, tsem, tosem):
    w = lax.axis_index("c") * _NS + lax.axis_index("s")
    b = w // _P
    p = w % _P
    tsk0 = _al(b * _TD + p * _C)
    tskd = _al(b * _ND + p * _C)
    src0 = _al(b * _SD + p * (_NT * _C))
    dst0 = _al(b * _ND + _TD + p * (_NT * _C))

    pltpu.async_copy(task.at[pl.ds(tsk0, _C)], tbuf, tsem)

    for u in range(_NBUF):
        pltpu.async_copy(text.at[pl.ds(_al(src0 + u * _C), _C)],
                         buf.at[u], isem.at[u])

    pltpu.make_async_copy(task.at[pl.ds(tsk0, _C)], tbuf, tsem).wait()
    pltpu.async_copy(tbuf, oe.at[pl.ds(tskd, _C)], tosem)

    @pl.when(p == 0)
    def _mask_labels():
        for q in range(_T // 16):
            pbuf_m[pl.ds(q * 16, 16)] = jnp.ones((16,), jnp.int32)
            pbuf_l[pl.ds(q * 16, 16)] = jnp.full((16,), -100, jnp.int32)
        mrow = pl.multiple_of(b * _S, 8)
        orow = pl.multiple_of(b * _N, 8)
        orow_t = pl.multiple_of(b * _N + _T, 8)
        pltpu.sync_copy(pbuf_m, om.at[pl.ds(orow, _T)])
        pltpu.sync_copy(pbuf_l, ol.at[pl.ds(orow, _T)])
        pltpu.sync_copy(mask.at[pl.ds(mrow, _S)], mbuf)
        pltpu.sync_copy(mbuf, om.at[pl.ds(orow_t, _S)])
        pltpu.sync_copy(labels.at[pl.ds(mrow, _S)], lbuf)
        pltpu.sync_copy(lbuf, ol.at[pl.ds(orow_t, _S)])

    def outer(g, carry):
        for u in range(_NBUF):
            j = g * _NBUF + u
            pltpu.make_async_copy(text.at[pl.ds(_al(src0 + j * _C), _C)],
                                  buf.at[u], isem.at[u]).wait()
            pltpu.async_copy(buf.at[u], oe.at[pl.ds(_al(dst0 + j * _C), _C)],
                             osem.at[u])
        for u in range(_NBUF):
            jn = (g + 1) * _NBUF + u

            @pl.when(jn < _NT)
            def _prefetch():
                pltpu.make_async_copy(
                    buf.at[u],
                    oe.at[pl.ds(_al(dst0 + (jn - _NBUF) * _C), _C)],
                    osem.at[u]).wait()
                pltpu.async_copy(text.at[pl.ds(_al(src0 + jn * _C), _C)],
                                 buf.at[u], isem.at[u])
        return carry

    lax.fori_loop(0, _NOUTER, outer, 0)

    for u in range(_NBUF):
        j = (_NOUTER - 1) * _NBUF + u
        pltpu.make_async_copy(buf.at[u], oe.at[pl.ds(_al(dst0 + j * _C), _C)],
                              osem.at[u]).wait()
    pltpu.make_async_copy(tbuf, oe.at[pl.ds(tskd, _C)], tosem).wait()


@jax.jit
def _sc_concat(text1, mask1, labels1, task1):
    mesh = plsc.VectorSubcoreMesh(core_axis_name="c", subcore_axis_name="s",
                                  num_cores=_NC, num_subcores=_NS)
    return pl.kernel(
        _sc_body,
        out_type=(
            jax.ShapeDtypeStruct((_B * _ND,), jnp.float32),
            jax.ShapeDtypeStruct((_B * _N,), jnp.int32),
            jax.ShapeDtypeStruct((_B * _N,), jnp.int32),
        ),
        mesh=mesh,
        scratch_types=(
            pltpu.VMEM((_NBUF, _C), jnp.float32),
            pltpu.VMEM((_C,), jnp.float32),
            pltpu.VMEM((_S,), jnp.int32),
            pltpu.VMEM((_S,), jnp.int32),
            pltpu.VMEM((_T,), jnp.int32),
            pltpu.VMEM((_T,), jnp.int32),
            pltpu.SemaphoreType.DMA((_NBUF,)),
            pltpu.SemaphoreType.DMA((_NBUF,)),
            pltpu.SemaphoreType.DMA,
            pltpu.SemaphoreType.DMA,
        ),
    )(text1, mask1, labels1, task1)


def kernel(text_embeds, attention_mask, labels, task_embeds):
    b, s, d = text_embeds.shape
    t = task_embeds.shape[1]
    assert (b, s, d, t) == (_B, _S, _D, _T)
    oe, om, ol = _sc_concat(
        text_embeds.reshape(-1),
        attention_mask.reshape(-1),
        labels.reshape(-1),
        task_embeds.reshape(-1),
    )
    return oe.reshape(b, t + s, d), om.reshape(b, t + s), ol.reshape(b, t + s)


# speedup vs baseline: 3.0079x; 3.0079x over previous
import jax
import jax.numpy as jnp
from jax import lax
from jax.experimental import pallas as pl
from jax.experimental.pallas import tpu as pltpu
from jax.experimental.pallas import tpu_sc as plsc

_B, _S, _D, _T = 4, 4096, 2048, 64
_N = _T + _S
_NC, _NS = 2, 16
_W = _NC * _NS
_P = _W // _B
_R = 8
_NT = _S // _P // _R
_NBUF = 4
_NOUTER = _NT // _NBUF


def _al(x):
    return pl.multiple_of(x, _R)


def _sc_body(text, task, oe, buf, tbuf, isem, osem, tsem, tosem):
    w = lax.axis_index("c") * _NS + lax.axis_index("s")
    b = w // _P
    p = w % _P
    src0 = p * (_NT * _R)
    dst0 = _T + src0

    pltpu.async_copy(task.at[b, pl.ds(_al(p * _R), _R), :], tbuf, tsem)

    for u in range(_NBUF):
        pltpu.async_copy(text.at[b, pl.ds(_al(src0 + u * _R), _R), :],
                         buf.at[u], isem.at[u])

    pltpu.make_async_copy(task.at[b, pl.ds(_al(p * _R), _R), :],
                          tbuf, tsem).wait()
    pltpu.async_copy(tbuf, oe.at[b, pl.ds(_al(p * _R), _R), :], tosem)

    def outer(g, carry):
        for u in range(_NBUF):
            j = g * _NBUF + u
            pltpu.make_async_copy(text.at[b, pl.ds(_al(src0 + j * _R), _R), :],
                                  buf.at[u], isem.at[u]).wait()
            pltpu.async_copy(buf.at[u],
                             oe.at[b, pl.ds(_al(dst0 + j * _R), _R), :],
                             osem.at[u])
        for u in range(_NBUF):
            jn = (g + 1) * _NBUF + u

            @pl.when(jn < _NT)
            def _prefetch():
                pltpu.make_async_copy(
                    buf.at[u],
                    oe.at[b, pl.ds(_al(dst0 + (jn - _NBUF) * _R), _R), :],
                    osem.at[u]).wait()
                pltpu.async_copy(text.at[b, pl.ds(_al(src0 + jn * _R), _R), :],
                                 buf.at[u], isem.at[u])
        return carry

    lax.fori_loop(0, _NOUTER, outer, 0)

    for u in range(_NBUF):
        j = (_NOUTER - 1) * _NBUF + u
        pltpu.make_async_copy(buf.at[u],
                              oe.at[b, pl.ds(_al(dst0 + j * _R), _R), :],
                              osem.at[u]).wait()
    pltpu.make_async_copy(tbuf, oe.at[b, pl.ds(_al(p * _R), _R), :],
                          tosem).wait()


def _mask_body(mask_ref, lab_ref, om_ref, ol_ref):
    nb, t = om_ref.shape[0], _T
    om_ref[...] = jnp.concatenate(
        [jnp.ones((nb, t), dtype=om_ref.dtype), mask_ref[...]], axis=1)
    ol_ref[...] = jnp.concatenate(
        [jnp.full((nb, t), -100, dtype=ol_ref.dtype), lab_ref[...]], axis=1)


@jax.jit
def _inject(text_embeds, attention_mask, labels, task_embeds):
    mesh = plsc.VectorSubcoreMesh(core_axis_name="c", subcore_axis_name="s",
                                  num_cores=_NC, num_subcores=_NS)
    oe = pl.kernel(
        _sc_body,
        out_type=jax.ShapeDtypeStruct((_B, _N, _D), jnp.float32),
        mesh=mesh,
        scratch_types=(
            pltpu.VMEM((_NBUF, _R, _D), jnp.float32),
            pltpu.VMEM((_R, _D), jnp.float32),
            pltpu.SemaphoreType.DMA((_NBUF,)),
            pltpu.SemaphoreType.DMA((_NBUF,)),
            pltpu.SemaphoreType.DMA,
            pltpu.SemaphoreType.DMA,
        ),
    )(text_embeds, task_embeds)
    om, ol = pl.pallas_call(
        _mask_body,
        out_shape=(
            jax.ShapeDtypeStruct((_B, _N), jnp.int32),
            jax.ShapeDtypeStruct((_B, _N), jnp.int32),
        ),
    )(attention_mask, labels)
    return oe, om, ol


def kernel(text_embeds, attention_mask, labels, task_embeds):
    b, s, d = text_embeds.shape
    t = task_embeds.shape[1]
    assert (b, s, d, t) == (_B, _S, _D, _T)
    return _inject(text_embeds, attention_mask, labels, task_embeds)
